# Initial kernel scaffold; baseline (speedup 1.0000x reference)
import jax
import jax.numpy as jnp
from jax.experimental import pallas as pl


def kernel(feat_p2, feat_p3, feat_p4, feat_p5, feat_p6, w_shared, b_shared, w_cls, b_cls, w_delta, b_delta):
    raise NotImplementedError("write your pallas kernel here")


# speedup vs baseline: 1.6192x; 1.6192x over previous
import functools

import jax
import jax.numpy as jnp
from jax.experimental import pallas as pl
from jax.experimental.pallas import tpu as pltpu

_C = 256
_F = 512


def _rpn_level_kernel(prev_ref, cent_ref, next_ref, w_ref, wcat_ref,
                      bsh_ref, bcat_ref, logits_ref, probs_ref, deltas_ref,
                      *, TH, W):
    prev = prev_ref[0]
    cent = cent_ref[0]
    nxt = next_ref[0]
    ext = jnp.concatenate([prev, cent, nxt], axis=0)

    acc = jnp.zeros((TH * W, _F), jnp.float32)
    for dy in range(3):
        for dx in range(3):
            x = ext[dy:dy + TH, dx:dx + W, :].reshape(TH * W, _C)
            acc += jnp.dot(x, w_ref[dy, dx],
                           preferred_element_type=jnp.float32)
    shared = jnp.maximum(acc + bsh_ref[...], 0.0).astype(jnp.bfloat16)

    y = jnp.dot(shared, wcat_ref[...],
                preferred_element_type=jnp.float32) + bcat_ref[...]
    logits = y[:, 0:6]
    deltas = y[:, 6:18]

    i = jax.lax.broadcasted_iota(jnp.int32, (6, 6), 0)
    j = jax.lax.broadcasted_iota(jnp.int32, (6, 6), 1)
    perm = ((i ^ 1) == j).astype(jnp.float32)
    swapped = jnp.dot(logits, perm, preferred_element_type=jnp.float32)
    m = jnp.maximum(logits, swapped)
    e = jnp.exp(logits - m)
    esw = jnp.exp(swapped - m)
    probs = e / (e + esw)

    logits_ref[0] = logits
    probs_ref[0] = probs
    deltas_ref[0] = deltas


def _run_level(feat, wsh, wcat, bsh, bcat, TH):
    B, H, W, C = feat.shape
    n_tiles = H // TH
    xp = jnp.pad(feat.astype(jnp.bfloat16),
                 ((0, 0), (TH, TH), (1, 1), (0, 0)))

    grid = (B, n_tiles)
    kfn = functools.partial(_rpn_level_kernel, TH=TH, W=W)
    logits, probs, deltas = pl.pallas_call(
        kfn,
        grid=grid,
        in_specs=[
            pl.BlockSpec((1, 1, W + 2, C),
                         lambda b, i: (b, (i + 1) * TH - 1, 0, 0)),
            pl.BlockSpec((1, TH, W + 2, C), lambda b, i: (b, i + 1, 0, 0)),
            pl.BlockSpec((1, 1, W + 2, C),
                         lambda b, i: (b, (i + 2) * TH, 0, 0)),
            pl.BlockSpec((3, 3, _C, _F), lambda b, i: (0, 0, 0, 0)),
            pl.BlockSpec((_F, 18), lambda b, i: (0, 0)),
            pl.BlockSpec((1, _F), lambda b, i: (0, 0)),
            pl.BlockSpec((1, 18), lambda b, i: (0, 0)),
        ],
        out_specs=[
            pl.BlockSpec((1, TH * W, 6), lambda b, i: (b, i, 0)),
            pl.BlockSpec((1, TH * W, 6), lambda b, i: (b, i, 0)),
            pl.BlockSpec((1, TH * W, 12), lambda b, i: (b, i, 0)),
        ],
        out_shape=[
            jax.ShapeDtypeStruct((B, H * W, 6), jnp.float32),
            jax.ShapeDtypeStruct((B, H * W, 6), jnp.float32),
            jax.ShapeDtypeStruct((B, H * W, 12), jnp.float32),
        ],
        compiler_params=pltpu.CompilerParams(
            dimension_semantics=("parallel", "parallel")),
    )(xp, xp, xp, wsh, wcat, bsh, bcat)
    return logits, probs, deltas


def kernel(feat_p2, feat_p3, feat_p4, feat_p5, feat_p6,
           w_shared, b_shared, w_cls, b_cls, w_delta, b_delta):
    feats = [feat_p2, feat_p3, feat_p4, feat_p5, feat_p6]
    tile_h = {256: 8, 128: 16, 64: 32, 32: 32, 16: 16}

    wsh = w_shared.astype(jnp.bfloat16)
    wcat = jnp.concatenate([w_cls[0, 0], w_delta[0, 0]],
                           axis=1).astype(jnp.bfloat16)
    bsh = b_shared.reshape(1, _F)
    bcat = jnp.concatenate([b_cls, b_delta]).reshape(1, 18)

    logits_list, probs_list, deltas_list = [], [], []
    for feat in feats:
        B, H, W, _ = feat.shape
        lg, pr, dl = _run_level(feat, wsh, wcat, bsh, bcat, tile_h[H])
        logits_list.append(lg.reshape(B, H * W * 3, 2))
        probs_list.append(pr.reshape(B, H * W * 3, 2))
        deltas_list.append(dl.reshape(B, H * W * 3, 4))

    return (jnp.concatenate(logits_list, axis=1),
            jnp.concatenate(probs_list, axis=1),
            jnp.concatenate(deltas_list, axis=1))


# speedup vs baseline: 2.1407x; 1.3221x over previous
import functools

import jax
import jax.numpy as jnp
from jax.experimental import pallas as pl
from jax.experimental.pallas import tpu as pltpu

_C = 256
_F = 512


def _rpn_level_kernel(*refs, TH, W, alias):
    if alias:
        (prev_ref, cent_ref, next_ref, w_ref, wcat_ref, bsh_ref, bcat_ref,
         _inL, _inP, _inD, logits_ref, probs_ref, deltas_ref) = refs
    else:
        (prev_ref, cent_ref, next_ref, w_ref, wcat_ref, bsh_ref, bcat_ref,
         logits_ref, probs_ref, deltas_ref) = refs

    prev = prev_ref[0]
    cent = cent_ref[0]
    nxt = next_ref[0]
    ext = jnp.concatenate([prev, cent, nxt], axis=0)

    cols = [ext[dy:dy + TH, dx:dx + W, :].reshape(TH * W, _C)
            for dy in range(3) for dx in range(3)]
    x = jnp.concatenate(cols, axis=1)
    acc = jnp.dot(x, w_ref[...], preferred_element_type=jnp.float32)
    shared = jnp.maximum(acc + bsh_ref[...], 0.0).astype(jnp.bfloat16)

    y = jnp.dot(shared, wcat_ref[...],
                preferred_element_type=jnp.float32) + bcat_ref[...]
    logits = y[:, 0:6]
    deltas = y[:, 6:18]

    i = jax.lax.broadcasted_iota(jnp.int32, (6, 6), 0)
    j = jax.lax.broadcasted_iota(jnp.int32, (6, 6), 1)
    perm = ((i ^ 1) == j).astype(jnp.float32)
    swapped = jnp.dot(logits, perm, preferred_element_type=jnp.float32)
    m = jnp.maximum(logits, swapped)
    e = jnp.exp(logits - m)
    esw = jnp.exp(swapped - m)
    probs = e / (e + esw)

    for a in range(3):
        logits_ref[0, a::3, :] = logits[:, 2 * a:2 * a + 2]
        probs_ref[0, a::3, :] = probs[:, 2 * a:2 * a + 2]
        deltas_ref[0, a::3, :] = deltas[:, 4 * a:4 * a + 4]


def _run_level(feat, wsh, wcat, bsh, bcat, TH, base, n_total, carry):
    B, H, W, C = feat.shape
    n_tiles = H // TH
    R = TH * W * 3
    base_blk = base // R
    xp = jnp.pad(feat.astype(jnp.bfloat16),
                 ((0, 0), (TH, TH), (1, 1), (0, 0)))

    alias = carry is not None
    grid = (B, n_tiles)
    kfn = functools.partial(_rpn_level_kernel, TH=TH, W=W, alias=alias)

    in_specs = [
        pl.BlockSpec((1, 1, W + 2, C),
                     lambda b, i: (b, (i + 1) * TH - 1, 0, 0)),
        pl.BlockSpec((1, TH, W + 2, C), lambda b, i: (b, i + 1, 0, 0)),
        pl.BlockSpec((1, 1, W + 2, C),
                     lambda b, i: (b, (i + 2) * TH, 0, 0)),
        pl.BlockSpec((9 * _C, _F), lambda b, i: (0, 0)),
        pl.BlockSpec((_F, 18), lambda b, i: (0, 0)),
        pl.BlockSpec((1, _F), lambda b, i: (0, 0)),
        pl.BlockSpec((1, 18), lambda b, i: (0, 0)),
    ]
    kwargs = {}
    if alias:
        in_specs += [pl.BlockSpec(memory_space=pltpu.MemorySpace.HBM)] * 3
        kwargs['input_output_aliases'] = {7: 0, 8: 1, 9: 2}

    def omap(b, i, bb=base_blk):
        return (b, bb + i, 0)

    logits, probs, deltas = pl.pallas_call(
        kfn,
        grid=grid,
        in_specs=in_specs,
        out_specs=[
            pl.BlockSpec((1, R, 2), omap),
            pl.BlockSpec((1, R, 2), omap),
            pl.BlockSpec((1, R, 4), omap),
        ],
        out_shape=[
            jax.ShapeDtypeStruct((B, n_total, 2), jnp.float32),
            jax.ShapeDtypeStruct((B, n_total, 2), jnp.float32),
            jax.ShapeDtypeStruct((B, n_total, 4), jnp.float32),
        ],
        compiler_params=pltpu.CompilerParams(
            dimension_semantics=("parallel", "parallel")),
        **kwargs,
    )(xp, xp, xp, wsh, wcat, bsh, bcat, *(carry if alias else ()))
    return logits, probs, deltas


def kernel(feat_p2, feat_p3, feat_p4, feat_p5, feat_p6,
           w_shared, b_shared, w_cls, b_cls, w_delta, b_delta):
    feats = [feat_p2, feat_p3, feat_p4, feat_p5, feat_p6]
    tile_h = {256: 8, 128: 16, 64: 32, 32: 32, 16: 16}
    n_total = 3 * sum(f.shape[1] * f.shape[2] for f in feats)

    wsh = w_shared.astype(jnp.bfloat16).reshape(9 * _C, _F)
    wcat = jnp.concatenate([w_cls[0, 0], w_delta[0, 0]],
                           axis=1).astype(jnp.bfloat16)
    bsh = b_shared.reshape(1, _F)
    bcat = jnp.concatenate([b_cls, b_delta]).reshape(1, 18)

    carry = None
    base = 0
    for feat in feats:
        _, H, W, _ = feat.shape
        carry = _run_level(feat, wsh, wcat, bsh, bcat, tile_h[H],
                           base, n_total, carry)
        base += H * W * 3

    return carry
